# baseline (speedup 1.0000x reference)
import functools

import jax
import jax.numpy as jnp
from jax import lax
from jax.experimental import pallas as pl
from jax.experimental.pallas import tpu as pltpu
from jax.experimental.pallas import tpu_sc as plsc

RATIO = 0.8
G = 64
N = 10000
NP = 10240
E = 320000
NC = 2
NS = 16
NW = NC * NS
CH = 128
NCH = 158
EP = NS * NCH * CH
FAST_C = 0
NCHD = 80
EPD = NW * NCHD * CH
FMIN = float(jnp.finfo(jnp.float32).min)


def _sc_scatter_kernel(src, rowi, coli, zeros, out,
                       rowall, colall, payload, sem, acc):
    c = lax.axis_index("c")
    s = lax.axis_index("s")
    rows_per_tile = NP // NS

    pltpu.sync_copy(zeros.at[pl.ds(s * rows_per_tile, rows_per_tile)],
                    acc.at[pl.ds(s * rows_per_tile, rows_per_tile)])

    @pl.when(c == FAST_C)
    def _():
        pltpu.sync_copy(rowi.at[s], rowall)
        pltpu.sync_copy(coli.at[s], colall)
        plsc.subcore_barrier()

        def chunk(ch, carry):
            pltpu.async_copy(src.at[rowall.at[ch]], payload, sem).wait()
            pltpu.sync_copy(payload, acc.at[colall.at[ch]], add=True)
            return carry

        lax.fori_loop(0, NCH, chunk, 0)
        plsc.subcore_barrier()

    pltpu.sync_copy(acc.at[pl.ds(s * rows_per_tile, rows_per_tile)],
                    out.at[c, pl.ds(s * rows_per_tile, rows_per_tile)])


def _make_sc_scatter(wid_feat):
    mesh = plsc.VectorSubcoreMesh(core_axis_name="c", subcore_axis_name="s")
    return pl.kernel(
        _sc_scatter_kernel,
        out_type=jax.ShapeDtypeStruct((NC, NP, wid_feat), jnp.float32),
        mesh=mesh,
        scratch_types=[
            pltpu.VMEM((NCH, CH), jnp.int32),
            pltpu.VMEM((NCH, CH), jnp.int32),
            pltpu.VMEM((CH, wid_feat), jnp.float32),
            pltpu.SemaphoreType.DMA,
            pltpu.VMEM_SHARED((NP, wid_feat), jnp.float32),
        ],
        compiler_params=pltpu.CompilerParams(use_tc_tiling_on_sc=False),
    )


EW = NCHD * CH
RPT = NP // NS


def _sc_deg_kernel(alive_h, rowf, colf, out,
                   aliveb, rowall, colall, degp, tbuf, acc):
    c = lax.axis_index("c")
    s = lax.axis_index("s")
    w = c * NS + s

    pltpu.sync_copy(alive_h, aliveb)
    pltpu.sync_copy(rowf.at[w], rowall)
    pltpu.sync_copy(colf.at[w], colall)

    def zero(i, carry):
        degp[pl.ds(i * 16, 16)] = jnp.zeros((16,), jnp.float32)
        return carry

    lax.fori_loop(0, NP // 16, zero, 0)

    def body(i, carry):
        r = rowall[pl.ds(i * 16, 16)]
        a = plsc.load_gather(aliveb, [r])
        ci = colall[pl.ds(i * 16, 16)]
        plsc.addupdate_scatter(degp, [ci], a)
        return carry

    lax.fori_loop(0, EW // 16, body, 0)

    pltpu.sync_copy(degp, acc.at[s])
    plsc.subcore_barrier()
    for t in range(NS):
        pltpu.sync_copy(acc.at[t, pl.ds(s * RPT, RPT)], tbuf.at[t])

    def red(j, carry):
        v = tbuf[0, pl.ds(j * 16, 16)]
        for t in range(1, NS):
            v = v + tbuf[t, pl.ds(j * 16, 16)]
        degp[pl.ds(j * 16, 16)] = v
        return carry

    lax.fori_loop(0, RPT // 16, red, 0)
    pltpu.sync_copy(degp.at[pl.ds(0, RPT)], out.at[c, pl.ds(s * RPT, RPT)])


def _make_sc_deg():
    mesh = plsc.VectorSubcoreMesh(core_axis_name="c", subcore_axis_name="s")
    return pl.kernel(
        _sc_deg_kernel,
        out_type=jax.ShapeDtypeStruct((NC, NP), jnp.float32),
        mesh=mesh,
        scratch_types=[
            pltpu.VMEM((NP,), jnp.float32),
            pltpu.VMEM((EW,), jnp.int32),
            pltpu.VMEM((EW,), jnp.int32),
            pltpu.VMEM((NP,), jnp.float32),
            pltpu.VMEM((NS, RPT), jnp.float32),
            pltpu.VMEM_SHARED((NS, NP), jnp.float32),
        ],
        compiler_params=pltpu.CompilerParams(use_tc_tiling_on_sc=False,
                                             needs_layout_passes=False),
    )


def _tc_pre_body(h, deg2, W, Ws, b2d, y, hw):
    dege = deg2[0, :, 0:1] + deg2[1, :, 0:1]
    dis = lax.rsqrt(1.0 + dege)
    xw = jnp.dot(h[...], W[...], preferred_element_type=jnp.float32)
    y[...] = xw * dis
    hw[...] = jnp.dot(h[...], Ws[...],
                      preferred_element_type=jnp.float32) + b2d[...]


def _tc_pre(h, deg2, W, Ws, bs):
    d = h.shape[1]
    blk = 512
    nb = NP // blk
    return pl.pallas_call(
        _tc_pre_body,
        grid=(nb,),
        in_specs=[
            pl.BlockSpec((blk, d), lambda i: (i, 0)),
            pl.BlockSpec((NC, blk, 1), lambda i: (0, i, 0)),
            pl.BlockSpec((d, 64), lambda i: (0, 0)),
            pl.BlockSpec((d, 64), lambda i: (0, 0)),
            pl.BlockSpec((1, 64), lambda i: (0, 0)),
        ],
        out_specs=[
            pl.BlockSpec((blk, 64), lambda i: (i, 0)),
            pl.BlockSpec((blk, 64), lambda i: (i, 0)),
        ],
        out_shape=[
            jax.ShapeDtypeStruct((NP, 64), jnp.float32),
            jax.ShapeDtypeStruct((NP, 64), jnp.float32),
        ],
    )(h, deg2, W, Ws, bs.reshape(1, 64))


BLK = 512
NB = NP // BLK


def _tc_combine_body(acc2, y, hw, deg2, alive_c, p2d, b2d, h_out, score_out):
    dege = deg2[0, :, 0:1] + deg2[1, :, 0:1]
    dis = lax.rsqrt(1.0 + dege)
    alive = alive_c[...]
    out = dis * (acc2[0] + acc2[1] + y[...]) + b2d[...]
    h = jnp.maximum(out + hw[...], 0.0) * alive
    h_out[...] = h
    p = p2d[...]
    nrm = jnp.sqrt(jnp.sum(p * p))
    score_out[...] = jnp.tanh(
        jnp.dot(h, p, preferred_element_type=jnp.float32) / nrm)


def _tc_combine(acc2, y, hw, deg2, alive_col, p, b):
    return pl.pallas_call(
        _tc_combine_body,
        grid=(NB,),
        in_specs=[
            pl.BlockSpec((NC, BLK, 64), lambda i: (0, i, 0)),
            pl.BlockSpec((BLK, 64), lambda i: (i, 0)),
            pl.BlockSpec((BLK, 64), lambda i: (i, 0)),
            pl.BlockSpec((NC, BLK, 1), lambda i: (0, i, 0)),
            pl.BlockSpec((BLK, 1), lambda i: (i, 0)),
            pl.BlockSpec((64, 1), lambda i: (0, 0)),
            pl.BlockSpec((1, 64), lambda i: (0, 0)),
        ],
        out_specs=[
            pl.BlockSpec((BLK, 64), lambda i: (i, 0)),
            pl.BlockSpec((BLK, 1), lambda i: (i, 0)),
        ],
        out_shape=[
            jax.ShapeDtypeStruct((NP, 64), jnp.float32),
            jax.ShapeDtypeStruct((NP, 1), jnp.float32),
        ],
    )(acc2, y, hw, deg2, alive_col, p.reshape(64, 1), b.reshape(1, 64))


def _tc_bisect_body(score_r, batch_r, alive_r, keep_out):
    score = score_r[...]
    bits = lax.bitcast_convert_type(score, jnp.int32)
    ks = jnp.where(bits >= 0, bits,
                   jnp.bitwise_xor(bits, jnp.int32(0x7FFFFFFF)))
    segs = lax.broadcasted_iota(jnp.int32, (G, NP), 0)
    oh = (batch_r[...] == segs) & (alive_r[...] > 0)
    counts = jnp.sum(jnp.where(oh, 1, 0), axis=1, keepdims=True)
    k = jnp.ceil(RATIO * counts.astype(jnp.float32)).astype(jnp.int32)

    cnt_pos = jnp.sum(jnp.where(oh & (ks >= 0), 1, 0), axis=1, keepdims=True)
    t0 = jnp.where(cnt_pos >= k, jnp.int32(0), jnp.int32(-0x80000000))

    def bit_round(i, t):
        cand = t + jnp.left_shift(jnp.int32(1), 30 - i)
        cnt = jnp.sum(jnp.where(oh & (ks >= cand), 1, 0), axis=1,
                      keepdims=True)
        return jnp.where(cnt >= k, cand, t)

    t = lax.fori_loop(0, 31, bit_round, t0)
    keep_gn = oh & (ks >= t)
    keep_out[...] = jnp.max(jnp.where(keep_gn, 1.0, 0.0), axis=0,
                            keepdims=True)


def _tc_bisect(score_row, batch_row, alive_row):
    return pl.pallas_call(
        _tc_bisect_body,
        out_shape=jax.ShapeDtypeStruct((1, NP), jnp.float32),
    )(score_row, batch_row, alive_row)


def _tc_stats_body(h, score, keep, batch_c, hk_out,
                   gmp_out, ssum_out, cnt_out):
    i = pl.program_id(0)
    keepf = keep[...]
    hk = h[...] * (score[...] * keepf)
    hk_out[...] = hk
    segs = lax.broadcasted_iota(jnp.int32, (BLK, G), 1)
    m = jnp.where(batch_c[...] == segs, 1.0, 0.0) * keepf
    ssum = lax.dot_general(m, hk, dimension_numbers=(((0,), (0,)), ((), ())),
                           preferred_element_type=jnp.float32)
    cnt = lax.dot_general(m, jnp.ones((BLK, 1), jnp.float32),
                          dimension_numbers=(((0,), (0,)), ((), ())),
                          preferred_element_type=jnp.float32)
    rows = []
    for s in range(G):
        msk = m[:, s:s + 1] > 0.0
        rows.append(jnp.max(jnp.where(msk, hk, FMIN), axis=0, keepdims=True))
    gmp = jnp.concatenate(rows, axis=0)

    @pl.when(i == 0)
    def _():
        gmp_out[...] = gmp
        ssum_out[...] = ssum
        cnt_out[...] = cnt

    @pl.when(i > 0)
    def _():
        gmp_out[...] = jnp.maximum(gmp_out[...], gmp)
        ssum_out[...] = ssum_out[...] + ssum
        cnt_out[...] = cnt_out[...] + cnt


def _tc_stats(h, score_col, keep_col, batch_col):
    return pl.pallas_call(
        _tc_stats_body,
        grid=(NB,),
        in_specs=[
            pl.BlockSpec((BLK, 64), lambda i: (i, 0)),
            pl.BlockSpec((BLK, 1), lambda i: (i, 0)),
            pl.BlockSpec((BLK, 1), lambda i: (i, 0)),
            pl.BlockSpec((BLK, 1), lambda i: (i, 0)),
        ],
        out_specs=[
            pl.BlockSpec((BLK, 64), lambda i: (i, 0)),
            pl.BlockSpec((G, 64), lambda i: (0, 0)),
            pl.BlockSpec((G, 64), lambda i: (0, 0)),
            pl.BlockSpec((G, 1), lambda i: (0, 0)),
        ],
        out_shape=[
            jax.ShapeDtypeStruct((NP, 64), jnp.float32),
            jax.ShapeDtypeStruct((G, 64), jnp.float32),
            jax.ShapeDtypeStruct((G, 64), jnp.float32),
            jax.ShapeDtypeStruct((G, 1), jnp.float32),
        ],
    )(h, score_col, keep_col, batch_col)


def _tc_head_body(g1, s1, c1, g2, s2, c2, g3, s3, c3,
                  Wl1, bl1, Wl2, bl2, Wl3, bl3, out):
    def xl(g, s, c):
        gap = s[...] / jnp.maximum(c[...], 1.0)
        return jnp.concatenate([g[...], gap], axis=1)

    z = xl(g1, s1, c1) + xl(g2, s2, c2) + xl(g3, s3, c3)
    z = jnp.maximum(jnp.dot(z, Wl1[...],
                            preferred_element_type=jnp.float32) + bl1[...], 0.0)
    z = jnp.maximum(jnp.dot(z, Wl2[...],
                            preferred_element_type=jnp.float32) + bl2[...], 0.0)
    z = jnp.dot(z, Wl3[...], preferred_element_type=jnp.float32) + bl3[...]
    zmax = jnp.max(z, axis=-1, keepdims=True)
    zs = z - zmax
    lse = jnp.log(jnp.sum(jnp.exp(zs), axis=-1, keepdims=True))
    out[...] = zs - lse


def _tc_head(stats, Wl1, bl1, Wl2, bl2, Wl3, bl3):
    nc = Wl3.shape[1]
    flat = [a for st in stats for a in st]
    return pl.pallas_call(
        _tc_head_body,
        out_shape=jax.ShapeDtypeStruct((G, nc), jnp.float32),
    )(*flat, Wl1, bl1.reshape(1, -1), Wl2, bl2.reshape(1, -1),
      Wl3, bl3.reshape(1, nc))


@jax.jit
def kernel(x, edge_index, batch, W1, b1, p1, W2, b2, p2, W3, b3, p3,
           Ws1, bs1, Ws2, bs2, Ws3, bs3, Wl1, bl1, Wl2, bl2, Wl3, bl3):
    xpad = jnp.pad(x, ((0, NP - N), (0, 0)))
    batchp = jnp.pad(batch.astype(jnp.int32), (0, NP - N))
    row = jnp.pad(edge_index[0].astype(jnp.int32), (0, EP - E),
                  constant_values=N).reshape(NS, NCH, CH)
    col = jnp.pad(edge_index[1].astype(jnp.int32), (0, EP - E),
                  constant_values=N).reshape(NS, NCH, CH)
    rowf = jnp.pad(edge_index[0].astype(jnp.int32), (0, EPD - E),
                   constant_values=N).reshape(NW, EW)
    colf = jnp.pad(edge_index[1].astype(jnp.int32), (0, EPD - E),
                   constant_values=N).reshape(NW, EW)
    zeros64 = jnp.zeros((NP, 64), jnp.float32)
    alive_flat = jnp.where(jnp.arange(NP) < N, 1.0, 0.0)

    deg_k = _make_sc_deg()
    feat_k = _make_sc_scatter(64)

    batch_row = batchp.reshape(1, NP)
    batch_col = batchp.reshape(NP, 1)

    h = xpad
    stats = []
    for (W, bb, Ws, bs, p) in ((W1, b1, Ws1, bs1, p1),
                               (W2, b2, Ws2, bs2, p2),
                               (W3, b3, Ws3, bs3, p3)):
        deg2 = deg_k(alive_flat, rowf, colf).reshape(NC, NP, 1)
        y, hw = _tc_pre(h, deg2, W, Ws, bs)
        acc2 = feat_k(y, row, col, zeros64)
        hconv, score = _tc_combine(acc2, y, hw, deg2,
                                   alive_flat.reshape(NP, 1), p, bb)
        keep_row = _tc_bisect(score.reshape(1, NP), batch_row,
                              alive_flat.reshape(1, NP))
        h, gmp, ssum, cnt = _tc_stats(hconv, score,
                                      keep_row.reshape(NP, 1), batch_col)
        alive_flat = keep_row.reshape(NP)
        stats.append((gmp, ssum, cnt))

    return _tc_head(stats, Wl1, bl1, Wl2, bl2, Wl3, bl3)


# speedup vs baseline: 1.2562x; 1.2562x over previous
import functools

import jax
import jax.numpy as jnp
from jax import lax
from jax.experimental import pallas as pl
from jax.experimental.pallas import tpu as pltpu
from jax.experimental.pallas import tpu_sc as plsc

RATIO = 0.8
G = 64
N = 10000
NP = 10240
E = 320000
NC = 2
NS = 16
NW = NC * NS
CH = 128
NCH = 79
EP = NW * NCH * CH
NCHD = 80
EPD = NW * NCHD * CH
FMIN = float(jnp.finfo(jnp.float32).min)


def _sc_scatter_kernel(src, rowi, coli, zeros, out,
                       rowall, colall, payload, sem, acc):
    c = lax.axis_index("c")
    s = lax.axis_index("s")
    w = c * NS + s
    rows_per_tile = NP // NS

    pltpu.sync_copy(zeros.at[pl.ds(s * rows_per_tile, rows_per_tile)],
                    acc.at[pl.ds(s * rows_per_tile, rows_per_tile)])
    pltpu.sync_copy(rowi.at[w], rowall)
    pltpu.sync_copy(coli.at[w], colall)
    plsc.subcore_barrier()

    def chunk(ch, carry):
        pltpu.async_copy(src.at[rowall.at[ch]], payload, sem).wait()
        pltpu.sync_copy(payload, acc.at[colall.at[ch]], add=True)
        return carry

    lax.fori_loop(0, NCH, chunk, 0)
    plsc.subcore_barrier()
    pltpu.sync_copy(acc.at[pl.ds(s * rows_per_tile, rows_per_tile)],
                    out.at[c, pl.ds(s * rows_per_tile, rows_per_tile)])


def _make_sc_scatter(wid_feat):
    mesh = plsc.VectorSubcoreMesh(core_axis_name="c", subcore_axis_name="s")
    return pl.kernel(
        _sc_scatter_kernel,
        out_type=jax.ShapeDtypeStruct((NC, NP, wid_feat), jnp.float32),
        mesh=mesh,
        scratch_types=[
            pltpu.VMEM((NCH, CH), jnp.int32),
            pltpu.VMEM((NCH, CH), jnp.int32),
            pltpu.VMEM((CH, wid_feat), jnp.float32),
            pltpu.SemaphoreType.DMA,
            pltpu.VMEM_SHARED((NP, wid_feat), jnp.float32),
        ],
        compiler_params=pltpu.CompilerParams(use_tc_tiling_on_sc=False),
    )


EW = NCHD * CH
RPT = NP // NS


def _sc_deg_kernel(alive_h, rowf, colf, out,
                   aliveb, rowall, colall, degp, tbuf, acc):
    c = lax.axis_index("c")
    s = lax.axis_index("s")
    w = c * NS + s

    pltpu.sync_copy(alive_h, aliveb)
    pltpu.sync_copy(rowf.at[w], rowall)
    pltpu.sync_copy(colf.at[w], colall)

    def zero(i, carry):
        degp[pl.ds(i * 16, 16)] = jnp.zeros((16,), jnp.float32)
        return carry

    lax.fori_loop(0, NP // 16, zero, 0)

    def body(i, carry):
        r = rowall[pl.ds(i * 16, 16)]
        a = plsc.load_gather(aliveb, [r])
        ci = colall[pl.ds(i * 16, 16)]
        plsc.addupdate_scatter(degp, [ci], a)
        return carry

    lax.fori_loop(0, EW // 16, body, 0)

    pltpu.sync_copy(degp, acc.at[s])
    plsc.subcore_barrier()
    for t in range(NS):
        pltpu.sync_copy(acc.at[t, pl.ds(s * RPT, RPT)], tbuf.at[t])

    def red(j, carry):
        v = tbuf[0, pl.ds(j * 16, 16)]
        for t in range(1, NS):
            v = v + tbuf[t, pl.ds(j * 16, 16)]
        degp[pl.ds(j * 16, 16)] = v
        return carry

    lax.fori_loop(0, RPT // 16, red, 0)
    pltpu.sync_copy(degp.at[pl.ds(0, RPT)], out.at[c, pl.ds(s * RPT, RPT)])


def _make_sc_deg():
    mesh = plsc.VectorSubcoreMesh(core_axis_name="c", subcore_axis_name="s")
    return pl.kernel(
        _sc_deg_kernel,
        out_type=jax.ShapeDtypeStruct((NC, NP), jnp.float32),
        mesh=mesh,
        scratch_types=[
            pltpu.VMEM((NP,), jnp.float32),
            pltpu.VMEM((EW,), jnp.int32),
            pltpu.VMEM((EW,), jnp.int32),
            pltpu.VMEM((NP,), jnp.float32),
            pltpu.VMEM((NS, RPT), jnp.float32),
            pltpu.VMEM_SHARED((NS, NP), jnp.float32),
        ],
        compiler_params=pltpu.CompilerParams(use_tc_tiling_on_sc=False,
                                             needs_layout_passes=False),
    )


def _tc_pre_body(h, deg2, W, Ws, b2d, y, hw):
    dege = deg2[0, :, 0:1] + deg2[1, :, 0:1]
    dis = lax.rsqrt(1.0 + dege)
    xw = jnp.dot(h[...], W[...], preferred_element_type=jnp.float32)
    y[...] = xw * dis
    hw[...] = jnp.dot(h[...], Ws[...],
                      preferred_element_type=jnp.float32) + b2d[...]


def _tc_pre(h, deg2, W, Ws, bs):
    d = h.shape[1]
    blk = 512
    nb = NP // blk
    return pl.pallas_call(
        _tc_pre_body,
        grid=(nb,),
        in_specs=[
            pl.BlockSpec((blk, d), lambda i: (i, 0)),
            pl.BlockSpec((NC, blk, 1), lambda i: (0, i, 0)),
            pl.BlockSpec((d, 64), lambda i: (0, 0)),
            pl.BlockSpec((d, 64), lambda i: (0, 0)),
            pl.BlockSpec((1, 64), lambda i: (0, 0)),
        ],
        out_specs=[
            pl.BlockSpec((blk, 64), lambda i: (i, 0)),
            pl.BlockSpec((blk, 64), lambda i: (i, 0)),
        ],
        out_shape=[
            jax.ShapeDtypeStruct((NP, 64), jnp.float32),
            jax.ShapeDtypeStruct((NP, 64), jnp.float32),
        ],
    )(h, deg2, W, Ws, bs.reshape(1, 64))


BLK = 512
NB = NP // BLK


def _tc_combine_body(acc2, y, hw, deg2, alive_c, p2d, b2d, h_out, score_out):
    dege = deg2[0, :, 0:1] + deg2[1, :, 0:1]
    dis = lax.rsqrt(1.0 + dege)
    alive = alive_c[...]
    out = dis * (acc2[0] + acc2[1] + y[...]) + b2d[...]
    h = jnp.maximum(out + hw[...], 0.0) * alive
    h_out[...] = h
    p = p2d[...]
    nrm = jnp.sqrt(jnp.sum(p * p))
    score_out[...] = jnp.tanh(
        jnp.dot(h, p, preferred_element_type=jnp.float32) / nrm)


def _tc_combine(acc2, y, hw, deg2, alive_col, p, b):
    return pl.pallas_call(
        _tc_combine_body,
        grid=(NB,),
        in_specs=[
            pl.BlockSpec((NC, BLK, 64), lambda i: (0, i, 0)),
            pl.BlockSpec((BLK, 64), lambda i: (i, 0)),
            pl.BlockSpec((BLK, 64), lambda i: (i, 0)),
            pl.BlockSpec((NC, BLK, 1), lambda i: (0, i, 0)),
            pl.BlockSpec((BLK, 1), lambda i: (i, 0)),
            pl.BlockSpec((64, 1), lambda i: (0, 0)),
            pl.BlockSpec((1, 64), lambda i: (0, 0)),
        ],
        out_specs=[
            pl.BlockSpec((BLK, 64), lambda i: (i, 0)),
            pl.BlockSpec((BLK, 1), lambda i: (i, 0)),
        ],
        out_shape=[
            jax.ShapeDtypeStruct((NP, 64), jnp.float32),
            jax.ShapeDtypeStruct((NP, 1), jnp.float32),
        ],
    )(acc2, y, hw, deg2, alive_col, p.reshape(64, 1), b.reshape(1, 64))


def _tc_bisect_body(score_r, batch_r, alive_r, keep_out):
    score = score_r[...]
    bits = lax.bitcast_convert_type(score, jnp.int32)
    ks = jnp.where(bits >= 0, bits,
                   jnp.bitwise_xor(bits, jnp.int32(0x7FFFFFFF)))
    segs = lax.broadcasted_iota(jnp.int32, (G, NP), 0)
    oh = (batch_r[...] == segs) & (alive_r[...] > 0)
    counts = jnp.sum(jnp.where(oh, 1, 0), axis=1, keepdims=True)
    k = jnp.ceil(RATIO * counts.astype(jnp.float32)).astype(jnp.int32)

    cnt_pos = jnp.sum(jnp.where(oh & (ks >= 0), 1, 0), axis=1, keepdims=True)
    t0 = jnp.where(cnt_pos >= k, jnp.int32(0), jnp.int32(-0x80000000))

    def bit_round(i, t):
        cand = t + jnp.left_shift(jnp.int32(1), 30 - i)
        cnt = jnp.sum(jnp.where(oh & (ks >= cand), 1, 0), axis=1,
                      keepdims=True)
        return jnp.where(cnt >= k, cand, t)

    t = lax.fori_loop(0, 31, bit_round, t0)
    keep_gn = oh & (ks >= t)
    keep_out[...] = jnp.max(jnp.where(keep_gn, 1.0, 0.0), axis=0,
                            keepdims=True)


def _tc_bisect(score_row, batch_row, alive_row):
    return pl.pallas_call(
        _tc_bisect_body,
        out_shape=jax.ShapeDtypeStruct((1, NP), jnp.float32),
    )(score_row, batch_row, alive_row)


def _tc_stats_body(h, score, keep, batch_c, hk_out,
                   gmp_out, ssum_out, cnt_out):
    i = pl.program_id(0)
    keepf = keep[...]
    hk = h[...] * (score[...] * keepf)
    hk_out[...] = hk
    segs = lax.broadcasted_iota(jnp.int32, (BLK, G), 1)
    m = jnp.where(batch_c[...] == segs, 1.0, 0.0) * keepf
    ssum = lax.dot_general(m, hk, dimension_numbers=(((0,), (0,)), ((), ())),
                           preferred_element_type=jnp.float32)
    cnt = lax.dot_general(m, jnp.ones((BLK, 1), jnp.float32),
                          dimension_numbers=(((0,), (0,)), ((), ())),
                          preferred_element_type=jnp.float32)
    rows = []
    for s in range(G):
        msk = m[:, s:s + 1] > 0.0
        rows.append(jnp.max(jnp.where(msk, hk, FMIN), axis=0, keepdims=True))
    gmp = jnp.concatenate(rows, axis=0)

    @pl.when(i == 0)
    def _():
        gmp_out[...] = gmp
        ssum_out[...] = ssum
        cnt_out[...] = cnt

    @pl.when(i > 0)
    def _():
        gmp_out[...] = jnp.maximum(gmp_out[...], gmp)
        ssum_out[...] = ssum_out[...] + ssum
        cnt_out[...] = cnt_out[...] + cnt


def _tc_stats(h, score_col, keep_col, batch_col):
    return pl.pallas_call(
        _tc_stats_body,
        grid=(NB,),
        in_specs=[
            pl.BlockSpec((BLK, 64), lambda i: (i, 0)),
            pl.BlockSpec((BLK, 1), lambda i: (i, 0)),
            pl.BlockSpec((BLK, 1), lambda i: (i, 0)),
            pl.BlockSpec((BLK, 1), lambda i: (i, 0)),
        ],
        out_specs=[
            pl.BlockSpec((BLK, 64), lambda i: (i, 0)),
            pl.BlockSpec((G, 64), lambda i: (0, 0)),
            pl.BlockSpec((G, 64), lambda i: (0, 0)),
            pl.BlockSpec((G, 1), lambda i: (0, 0)),
        ],
        out_shape=[
            jax.ShapeDtypeStruct((NP, 64), jnp.float32),
            jax.ShapeDtypeStruct((G, 64), jnp.float32),
            jax.ShapeDtypeStruct((G, 64), jnp.float32),
            jax.ShapeDtypeStruct((G, 1), jnp.float32),
        ],
    )(h, score_col, keep_col, batch_col)


def _tc_head_body(g1, s1, c1, g2, s2, c2, g3, s3, c3,
                  Wl1, bl1, Wl2, bl2, Wl3, bl3, out):
    def xl(g, s, c):
        gap = s[...] / jnp.maximum(c[...], 1.0)
        return jnp.concatenate([g[...], gap], axis=1)

    z = xl(g1, s1, c1) + xl(g2, s2, c2) + xl(g3, s3, c3)
    z = jnp.maximum(jnp.dot(z, Wl1[...],
                            preferred_element_type=jnp.float32) + bl1[...], 0.0)
    z = jnp.maximum(jnp.dot(z, Wl2[...],
                            preferred_element_type=jnp.float32) + bl2[...], 0.0)
    z = jnp.dot(z, Wl3[...], preferred_element_type=jnp.float32) + bl3[...]
    zmax = jnp.max(z, axis=-1, keepdims=True)
    zs = z - zmax
    lse = jnp.log(jnp.sum(jnp.exp(zs), axis=-1, keepdims=True))
    out[...] = zs - lse


def _tc_head(stats, Wl1, bl1, Wl2, bl2, Wl3, bl3):
    nc = Wl3.shape[1]
    flat = [a for st in stats for a in st]
    return pl.pallas_call(
        _tc_head_body,
        out_shape=jax.ShapeDtypeStruct((G, nc), jnp.float32),
    )(*flat, Wl1, bl1.reshape(1, -1), Wl2, bl2.reshape(1, -1),
      Wl3, bl3.reshape(1, nc))


@jax.jit
def kernel(x, edge_index, batch, W1, b1, p1, W2, b2, p2, W3, b3, p3,
           Ws1, bs1, Ws2, bs2, Ws3, bs3, Wl1, bl1, Wl2, bl2, Wl3, bl3):
    xpad = jnp.pad(x, ((0, NP - N), (0, 0)))
    batchp = jnp.pad(batch.astype(jnp.int32), (0, NP - N))
    row = jnp.pad(edge_index[0].astype(jnp.int32), (0, EP - E),
                  constant_values=N).reshape(NW, NCH, CH)
    col = jnp.pad(edge_index[1].astype(jnp.int32), (0, EP - E),
                  constant_values=N).reshape(NW, NCH, CH)
    rowf = jnp.pad(edge_index[0].astype(jnp.int32), (0, EPD - E),
                   constant_values=N).reshape(NW, EW)
    colf = jnp.pad(edge_index[1].astype(jnp.int32), (0, EPD - E),
                   constant_values=N).reshape(NW, EW)
    zeros64 = jnp.zeros((NP, 64), jnp.float32)
    alive_flat = jnp.where(jnp.arange(NP) < N, 1.0, 0.0)

    deg_k = _make_sc_deg()
    feat_k = _make_sc_scatter(64)

    batch_row = batchp.reshape(1, NP)
    batch_col = batchp.reshape(NP, 1)

    h = xpad
    stats = []
    for (W, bb, Ws, bs, p) in ((W1, b1, Ws1, bs1, p1),
                               (W2, b2, Ws2, bs2, p2),
                               (W3, b3, Ws3, bs3, p3)):
        deg2 = deg_k(alive_flat, rowf, colf).reshape(NC, NP, 1)
        y, hw = _tc_pre(h, deg2, W, Ws, bs)
        acc2 = feat_k(y, row, col, zeros64)
        hconv, score = _tc_combine(acc2, y, hw, deg2,
                                   alive_flat.reshape(NP, 1), p, bb)
        keep_row = _tc_bisect(score.reshape(1, NP), batch_row,
                              alive_flat.reshape(1, NP))
        h, gmp, ssum, cnt = _tc_stats(hconv, score,
                                      keep_row.reshape(NP, 1), batch_col)
        alive_flat = keep_row.reshape(NP)
        stats.append((gmp, ssum, cnt))

    return _tc_head(stats, Wl1, bl1, Wl2, bl2, Wl3, bl3)
